# Initial kernel scaffold; baseline (speedup 1.0000x reference)
import functools

import jax
import jax.numpy as jnp
from jax import lax
from jax.experimental import pallas as pl
from jax.experimental.pallas import tpu as pltpu
from jax.experimental.pallas import tpu_sc as plsc

N = 10000
E = 320000
D = 128
H = 5
C = 128
HP = 16
NEG = 0.2
INVH = 1.0 / H

NW = 32
EW = E // NW
CH = 80
NCH = EW // CH
RPT = N // 16


def _s1_body(x_ref, w_ref, attm_ref, psum_ref, mask_ref,
             alphas_ref, exs_ref, g16_ref):
    wa = jnp.dot(w_ref[...], attm_ref[...], preferred_element_type=jnp.float32)
    al = jnp.dot(x_ref[...], wa, preferred_element_type=jnp.float32)
    alphas_ref[...] = al
    asum = jnp.dot(al, psum_ref[...], preferred_element_type=jnp.float32)
    gmax = jnp.max(al, axis=0, keepdims=True)
    gsum = jnp.dot(gmax, psum_ref[...], preferred_element_type=jnp.float32)
    g = jnp.where(gsum >= 0.0, gsum, NEG * gsum)
    lr = jnp.where(asum >= 0.0, asum, NEG * asum)
    exs_ref[...] = jnp.exp(lr - g) * mask_ref[...]
    g16_ref[...] = g


def _s1(x, w, attm, psum, mask):
    return pl.pallas_call(
        _s1_body,
        out_shape=[
            jax.ShapeDtypeStruct((N, HP), jnp.float32),
            jax.ShapeDtypeStruct((N, HP), jnp.float32),
            jax.ShapeDtypeStruct((1, HP), jnp.float32),
        ],
    )(x, w, attm, psum, mask)


def _s2_body(x_ref, w_ref, xw_ref):
    xw_ref[...] = jnp.dot(x_ref[...], w_ref[...],
                          preferred_element_type=jnp.float32)


def _s2(x, w):
    blk = 1000
    return pl.pallas_call(
        _s2_body,
        grid=(N // blk,),
        in_specs=[
            pl.BlockSpec((blk, D), lambda i: (i, 0)),
            pl.BlockSpec((D, H * C), lambda i: (0, 0)),
        ],
        out_specs=pl.BlockSpec((blk, H * C), lambda i: (i, 0)),
        out_shape=jax.ShapeDtypeStruct((N, H * C), jnp.float32),
    )(x, w)


def _s3_body(pa_ref, exs_ref, den_ref, selfw_ref):
    den = pa_ref[0] + pa_ref[1] + exs_ref[...]
    den_ref[...] = den
    selfw_ref[...] = exs_ref[...] / (den + 1e-16) * INVH


def _s3(pa, exs):
    return pl.pallas_call(
        _s3_body,
        out_shape=[
            jax.ShapeDtypeStruct((N, HP), jnp.float32),
            jax.ShapeDtypeStruct((N, HP), jnp.float32),
        ],
    )(pa, exs)


def _s4_body(x_ref, p0_ref, p1_ref, xw_ref, selfw_ref, r_ref, bias_ref,
             out_ref):
    sw = jnp.dot(selfw_ref[...], r_ref[...],
                 preferred_element_type=jnp.float32)
    prod = sw * xw_ref[...]
    s = prod[:, 0:C]
    for h in range(1, H):
        s = s + prod[:, h * C:(h + 1) * C]
    out_ref[...] = x_ref[...] + p0_ref[...] + p1_ref[...] + s + bias_ref[...]


def _s4(x, p0, p1, xw, selfw, r, bias):
    blk = 1000
    return pl.pallas_call(
        _s4_body,
        grid=(N // blk,),
        in_specs=[
            pl.BlockSpec((blk, C), lambda i: (i, 0)),
            pl.BlockSpec((blk, C), lambda i: (i, 0)),
            pl.BlockSpec((blk, C), lambda i: (i, 0)),
            pl.BlockSpec((blk, H * C), lambda i: (i, 0)),
            pl.BlockSpec((blk, HP), lambda i: (i, 0)),
            pl.BlockSpec((HP, H * C), lambda i: (0, 0)),
            pl.BlockSpec((1, C), lambda i: (0, 0)),
        ],
        out_specs=pl.BlockSpec((blk, C), lambda i: (i, 0)),
        out_shape=jax.ShapeDtypeStruct((N, C), jnp.float32),
    )(x, p0, p1, xw, selfw, r, bias)


def _pa_body(alphas_hbm, g_hbm, src_hbm, dst_hbm, pout_hbm,
             gbuf, sbuf, dbuf, asg, adg, stage, zb, tbuf, den, sem):
    cid = lax.axis_index("c")
    sid = lax.axis_index("s")
    wid = cid * 16 + sid
    pltpu.sync_copy(g_hbm, gbuf)

    z = jnp.zeros((16,), jnp.float32)

    def _zrow(buf, r):
        def body(i, carry):
            buf[i, :] = z
            return carry
        lax.fori_loop(0, r, body, 0)

    _zrow(zb, 125)
    _zrow(stage, CH)
    for k in range(5):
        pltpu.sync_copy(zb, den.at[pl.ds(sid * RPT + k * 125, 125)])
    plsc.subcore_barrier()

    iota = lax.iota(jnp.int32, 16)
    gvecs = [plsc.load_gather(gbuf, [jnp.full((16,), h, jnp.int32)])
             for h in range(H)]

    def chunk(c, carry):
        base = wid * EW + c * CH
        pltpu.sync_copy(src_hbm.at[pl.ds(base, CH)], sbuf)
        pltpu.sync_copy(dst_hbm.at[pl.ds(base, CH)], dbuf)
        pltpu.async_copy(alphas_hbm.at[sbuf], asg, sem).wait()
        pltpu.async_copy(alphas_hbm.at[dbuf], adg, sem).wait()
        for grp in range(CH // 16):
            e_idx = iota + grp * 16
            for h in range(H):
                h_s = jnp.full((16,), h, jnp.int32)
                h_d = jnp.full((16,), 8 + h, jnp.int32)
                a_s = plsc.load_gather(asg, [e_idx, h_s])
                a_d = plsc.load_gather(adg, [e_idx, h_d])
                al = a_s + a_d
                al = jnp.where(al >= 0.0, al, NEG * al)
                ex = jnp.exp(al - gvecs[h])
                plsc.store_scatter(stage, [e_idx, h_s], ex)
        pltpu.sync_copy(stage, den.at[dbuf], add=True)
        return carry

    lax.fori_loop(0, NCH, chunk, 0)
    plsc.subcore_barrier()
    pltpu.sync_copy(den.at[pl.ds(sid * RPT, RPT)], tbuf)
    pltpu.sync_copy(tbuf, pout_hbm.at[cid, pl.ds(sid * RPT, RPT)])


def _pass_a(alphas, g16, src, dst):
    kfn = pl.kernel(
        _pa_body,
        out_type=jax.ShapeDtypeStruct((2, N, HP), jnp.float32),
        mesh=plsc.VectorSubcoreMesh(core_axis_name="c", subcore_axis_name="s"),
        scratch_types=[
            pltpu.VMEM((16,), jnp.float32),
            pltpu.VMEM((CH,), jnp.int32),
            pltpu.VMEM((CH,), jnp.int32),
            pltpu.VMEM((CH, HP), jnp.float32),
            pltpu.VMEM((CH, HP), jnp.float32),
            pltpu.VMEM((CH, HP), jnp.float32),
            pltpu.VMEM((125, HP), jnp.float32),
            pltpu.VMEM((RPT, HP), jnp.float32),
            pltpu.VMEM_SHARED((N, HP), jnp.float32),
            pltpu.SemaphoreType.DMA,
        ],
    )
    return kfn(alphas, g16, src, dst)


def _pb_body(alphas_hbm, g_hbm, den_hbm, xw_hbm, src_hbm, dst_hbm, pout_hbm,
             gbuf, sbuf, dbuf, asg, adg, dng, wst, xwg, msg, zb, acc, sem):
    cid = lax.axis_index("c")
    sid = lax.axis_index("s")
    wid = cid * 16 + sid
    pltpu.sync_copy(g_hbm, gbuf)

    z = jnp.zeros((16,), jnp.float32)

    def zbody(i, carry):
        zb[i // 8, pl.ds((i % 8) * 16, 16)] = z
        return carry
    lax.fori_loop(0, 125 * 8, zbody, 0)
    for k in range(5):
        pltpu.sync_copy(zb, acc.at[pl.ds(sid * RPT + k * 125, 125)])
    plsc.subcore_barrier()

    iota = lax.iota(jnp.int32, 16)
    gvecs = [plsc.load_gather(gbuf, [jnp.full((16,), h, jnp.int32)])
             for h in range(H)]

    def chunk(c, carry):
        base = wid * EW + c * CH
        pltpu.sync_copy(src_hbm.at[pl.ds(base, CH)], sbuf)
        pltpu.sync_copy(dst_hbm.at[pl.ds(base, CH)], dbuf)
        pltpu.async_copy(alphas_hbm.at[sbuf], asg, sem).wait()
        pltpu.async_copy(alphas_hbm.at[dbuf], adg, sem).wait()
        pltpu.async_copy(den_hbm.at[dbuf], dng, sem).wait()
        pltpu.async_copy(xw_hbm.at[sbuf], xwg, sem).wait()
        for grp in range(CH // 16):
            e_idx = iota + grp * 16
            for h in range(H):
                h_s = jnp.full((16,), h, jnp.int32)
                a_s = plsc.load_gather(asg, [e_idx, h_s])
                a_d = plsc.load_gather(adg, [e_idx, jnp.full((16,), 8 + h,
                                                             jnp.int32)])
                dn = plsc.load_gather(dng, [e_idx, h_s])
                al = a_s + a_d
                al = jnp.where(al >= 0.0, al, NEG * al)
                ex = jnp.exp(al - gvecs[h])
                w = ex / (dn + 1e-16) * INVH
                plsc.store_scatter(wst, [e_idx, h_s], w)

        def edge(e, carry2):
            wv = [plsc.load_gather(wst, [jnp.full((16,), e, jnp.int32),
                                         jnp.full((16,), h, jnp.int32)])
                  for h in range(H)]
            for cb in range(C // 16):
                a = wv[0] * xwg[e, pl.ds(cb * 16, 16)]
                for h in range(1, H):
                    a = a + wv[h] * xwg[e, pl.ds(h * C + cb * 16, 16)]
                msg[e, pl.ds(cb * 16, 16)] = a
            return carry2

        lax.fori_loop(0, CH, edge, 0)
        pltpu.sync_copy(msg, acc.at[dbuf], add=True)
        return carry

    lax.fori_loop(0, NCH, chunk, 0)
    plsc.subcore_barrier()
    for k in range(5):
        pltpu.sync_copy(acc.at[pl.ds(sid * RPT + k * 125, 125)], zb)
        pltpu.sync_copy(zb, pout_hbm.at[cid, pl.ds(sid * RPT + k * 125, 125)])


def _pass_b(alphas, g16, den, xw, src, dst):
    kfn = pl.kernel(
        _pb_body,
        out_type=jax.ShapeDtypeStruct((2, N, C), jnp.float32),
        mesh=plsc.VectorSubcoreMesh(core_axis_name="c", subcore_axis_name="s"),
        scratch_types=[
            pltpu.VMEM((16,), jnp.float32),
            pltpu.VMEM((CH,), jnp.int32),
            pltpu.VMEM((CH,), jnp.int32),
            pltpu.VMEM((CH, HP), jnp.float32),
            pltpu.VMEM((CH, HP), jnp.float32),
            pltpu.VMEM((CH, HP), jnp.float32),
            pltpu.VMEM((CH, 8), jnp.float32),
            pltpu.VMEM((CH, H * C), jnp.float32),
            pltpu.VMEM((CH, C), jnp.float32),
            pltpu.VMEM((125, C), jnp.float32),
            pltpu.VMEM_SHARED((N, C), jnp.float32),
            pltpu.SemaphoreType.DMA,
        ],
    )
    return kfn(alphas, g16, den, xw, src, dst)


def kernel(features, edge_index, W, att_src, att_dst, bias):
    f32 = jnp.float32
    eye5 = jnp.eye(H, dtype=f32)
    src_cols = (att_src[:, :, None] * eye5[:, None, :]).reshape(H * C, H)
    dst_cols = (att_dst[:, :, None] * eye5[:, None, :]).reshape(H * C, H)
    zpad = jnp.zeros((H * C, 3), f32)
    attm = jnp.concatenate([src_cols, zpad, dst_cols, zpad], axis=1)
    psum = jnp.zeros((HP, HP), f32)
    idx = jnp.arange(H)
    psum = psum.at[idx, idx].set(1.0).at[idx + 8, idx].set(1.0)
    mask = jnp.concatenate(
        [jnp.ones((1, H), f32), jnp.zeros((1, HP - H), f32)], axis=1)
    r = jnp.concatenate(
        [jnp.repeat(jnp.eye(H, dtype=f32), C, axis=1),
         jnp.zeros((HP - H, H * C), f32)], axis=0)

    src = edge_index[0]
    dst = edge_index[1]

    alphas, exs, g16 = _s1(features, W, attm, psum, mask)
    xw = _s2(features, W)
    pa = _pass_a(alphas, g16.reshape(HP), src, dst)
    den, selfw = _s3(pa, exs)
    pb = _pass_b(alphas, g16.reshape(HP), den, xw, src, dst)
    return _s4(features, pb[0], pb[1], xw, selfw, r, bias.reshape(1, C))


# speedup vs baseline: 17.2688x; 17.2688x over previous
import jax
import jax.numpy as jnp
from jax import lax
from jax.experimental import pallas as pl
from jax.experimental.pallas import tpu as pltpu
from jax.experimental.pallas import tpu_sc as plsc

N = 10000
E = 320000
D = 128
H = 5
C = 128
HP = 16
NEG = 0.2
INVH = 1.0 / H

NW = 32
EW = E // NW
CHA = 80
NCHA = EW // CHA
CHB = 40
NCHB = EW // CHB
RPT = 624
RCHA, NZA = 104, 6
RCHB, NZB = 48, 13


def _s1_body(x_ref, w_ref, attm_ref, psum_ref, mask_ref,
             alphas_ref, exs_ref, g16_ref):
    wa = jnp.dot(w_ref[...], attm_ref[...], preferred_element_type=jnp.float32)
    al = jnp.dot(x_ref[...], wa, preferred_element_type=jnp.float32)
    alphas_ref[...] = al
    asum = jnp.dot(al, psum_ref[...], preferred_element_type=jnp.float32)
    gmax = jnp.max(al, axis=0, keepdims=True)
    gsum = jnp.dot(gmax, psum_ref[...], preferred_element_type=jnp.float32)
    g = jnp.where(gsum >= 0.0, gsum, NEG * gsum)
    lr = jnp.where(asum >= 0.0, asum, NEG * asum)
    exs_ref[...] = jnp.exp(lr - g) * mask_ref[...]
    g16_ref[...] = g


def _s1(x, w, attm, psum, mask):
    return pl.pallas_call(
        _s1_body,
        out_shape=[
            jax.ShapeDtypeStruct((N, HP), jnp.float32),
            jax.ShapeDtypeStruct((N, HP), jnp.float32),
            jax.ShapeDtypeStruct((1, HP), jnp.float32),
        ],
    )(x, w, attm, psum, mask)


def _s2_body(x_ref, w_ref, xw_ref):
    xw_ref[...] = jnp.dot(x_ref[...], w_ref[...],
                          preferred_element_type=jnp.float32)


def _s2(x, w):
    blk = 1000
    return pl.pallas_call(
        _s2_body,
        grid=(N // blk,),
        in_specs=[
            pl.BlockSpec((blk, D), lambda i: (i, 0)),
            pl.BlockSpec((D, H * C), lambda i: (0, 0)),
        ],
        out_specs=pl.BlockSpec((blk, H * C), lambda i: (i, 0)),
        out_shape=jax.ShapeDtypeStruct((N, H * C), jnp.float32),
    )(x, w)


def _s3_body(pa_ref, exs_ref, den_ref, selfw_ref):
    den = pa_ref[0] + pa_ref[1] + exs_ref[...]
    den_ref[...] = den
    selfw_ref[...] = exs_ref[...] / (den + 1e-16) * INVH


def _s3(pa, exs):
    return pl.pallas_call(
        _s3_body,
        out_shape=[
            jax.ShapeDtypeStruct((N, HP), jnp.float32),
            jax.ShapeDtypeStruct((N, HP), jnp.float32),
        ],
    )(pa, exs)


def _s4_body(x_ref, p0_ref, p1_ref, xw_ref, selfw_ref, r_ref, bias_ref,
             out_ref):
    sw = jnp.dot(selfw_ref[...], r_ref[...],
                 preferred_element_type=jnp.float32)
    prod = sw * xw_ref[...]
    s = prod[:, 0:C]
    for h in range(1, H):
        s = s + prod[:, h * C:(h + 1) * C]
    out_ref[...] = x_ref[...] + p0_ref[...] + p1_ref[...] + s + bias_ref[...]


def _s4(x, p0, p1, xw, selfw, r, bias):
    blk = 1000
    return pl.pallas_call(
        _s4_body,
        grid=(N // blk,),
        in_specs=[
            pl.BlockSpec((blk, C), lambda i: (i, 0)),
            pl.BlockSpec((blk, C), lambda i: (i, 0)),
            pl.BlockSpec((blk, C), lambda i: (i, 0)),
            pl.BlockSpec((blk, H * C), lambda i: (i, 0)),
            pl.BlockSpec((blk, HP), lambda i: (i, 0)),
            pl.BlockSpec((HP, H * C), lambda i: (0, 0)),
            pl.BlockSpec((1, C), lambda i: (0, 0)),
        ],
        out_specs=pl.BlockSpec((blk, C), lambda i: (i, 0)),
        out_shape=jax.ShapeDtypeStruct((N, C), jnp.float32),
    )(x, p0, p1, xw, selfw, r, bias)


def _lane_consts():
    iota = lax.iota(jnp.int32, 16)
    shift = jnp.bitwise_and(iota + 8, 15)
    maskv = jnp.where(iota < H, 1.0, 0.0).astype(jnp.float32)
    return iota, shift, maskv


def _edge_ex(asr, adr, shift, maskv, gv):
    al = asr + adr.at[shift].get(mode="promise_in_bounds")
    al = jnp.where(al >= 0.0, al, NEG * al)
    return jnp.exp(al - gv) * maskv


def _pa_body(alphas_hbm, g_hbm, src_hbm, dst_hbm, pout_hbm,
             gbuf, sbuf, dbuf, asg, adg, stage, zb, den, sem):
    cid = lax.axis_index("c")
    sid = lax.axis_index("s")
    wid = cid * 16 + sid
    pltpu.sync_copy(g_hbm, gbuf)

    z = jnp.zeros((16,), jnp.float32)

    def _zrow(buf, r):
        def body(i, carry):
            buf[i, :] = z
            return carry
        lax.fori_loop(0, r, body, 0)

    _zrow(zb, RCHA)
    for k in range(NZA):
        pltpu.sync_copy(zb, den.at[pl.ds(sid * RPT + k * RCHA, RCHA)])

    @pl.when(sid == 15)
    def _():
        pltpu.sync_copy(zb.at[pl.ds(0, 16)], den.at[pl.ds(16 * RPT, 16)])
    plsc.subcore_barrier()

    _, shift, maskv = _lane_consts()
    gv = gbuf[...]

    def chunk(c, carry):
        base = wid * EW + c * CHA
        pltpu.sync_copy(src_hbm.at[pl.ds(base, CHA)], sbuf)
        pltpu.sync_copy(dst_hbm.at[pl.ds(base, CHA)], dbuf)
        pltpu.async_copy(alphas_hbm.at[sbuf], asg, sem).wait()
        pltpu.async_copy(alphas_hbm.at[dbuf], adg, sem).wait()

        def edge(e, carry2):
            stage[e, :] = _edge_ex(asg[e, :], adg[e, :], shift, maskv, gv)
            return carry2

        lax.fori_loop(0, CHA, edge, 0)
        pltpu.sync_copy(stage, den.at[dbuf], add=True)
        return carry

    lax.fori_loop(0, NCHA, chunk, 0)
    plsc.subcore_barrier()
    for k in range(NZA):
        pltpu.sync_copy(den.at[pl.ds(sid * RPT + k * RCHA, RCHA)], zb)
        pltpu.sync_copy(zb, pout_hbm.at[cid, pl.ds(sid * RPT + k * RCHA, RCHA)])

    @pl.when(sid == 15)
    def _():
        pltpu.sync_copy(den.at[pl.ds(16 * RPT, 16)], zb.at[pl.ds(0, 16)])
        pltpu.sync_copy(zb.at[pl.ds(0, 16)],
                        pout_hbm.at[cid, pl.ds(16 * RPT, 16)])


def _pass_a(alphas, g16, src, dst):
    kfn = pl.kernel(
        _pa_body,
        out_type=jax.ShapeDtypeStruct((2, N, HP), jnp.float32),
        mesh=plsc.VectorSubcoreMesh(core_axis_name="c", subcore_axis_name="s"),
        compiler_params=pltpu.CompilerParams(use_tc_tiling_on_sc=False),
        scratch_types=[
            pltpu.VMEM((HP,), jnp.float32),
            pltpu.VMEM((CHA,), jnp.int32),
            pltpu.VMEM((CHA,), jnp.int32),
            pltpu.VMEM((CHA, HP), jnp.float32),
            pltpu.VMEM((CHA, HP), jnp.float32),
            pltpu.VMEM((CHA, HP), jnp.float32),
            pltpu.VMEM((RCHA, HP), jnp.float32),
            pltpu.VMEM_SHARED((N, HP), jnp.float32),
            pltpu.SemaphoreType.DMA,
        ],
    )
    return kfn(alphas, g16, src, dst)


def _pb_body(alphas_hbm, g_hbm, den_hbm, xw_hbm, src_hbm, dst_hbm, pout_hbm,
             gbuf, sbuf, dbuf, asg, adg, dng, xwg, msg, zb, acc, sem):
    cid = lax.axis_index("c")
    sid = lax.axis_index("s")
    wid = cid * 16 + sid
    pltpu.sync_copy(g_hbm, gbuf)

    z = jnp.zeros((16,), jnp.float32)

    def zbody(i, carry):
        zb[i // 8, pl.ds((i % 8) * 16, 16)] = z
        return carry
    lax.fori_loop(0, RCHB * 8, zbody, 0)
    for k in range(NZB):
        pltpu.sync_copy(zb, acc.at[pl.ds(sid * RPT + k * RCHB, RCHB)])

    @pl.when(sid == 15)
    def _():
        pltpu.sync_copy(zb.at[pl.ds(0, 16)], acc.at[pl.ds(16 * RPT, 16)])
    plsc.subcore_barrier()

    _, shift, maskv = _lane_consts()
    gv = gbuf[...]
    hidx = [jnp.full((16,), h, jnp.int32) for h in range(H)]

    def chunk(c, carry):
        base = wid * EW + c * CHB
        pltpu.sync_copy(src_hbm.at[pl.ds(base, CHB)], sbuf)
        pltpu.sync_copy(dst_hbm.at[pl.ds(base, CHB)], dbuf)
        pltpu.async_copy(alphas_hbm.at[sbuf], asg, sem).wait()
        pltpu.async_copy(alphas_hbm.at[dbuf], adg, sem).wait()
        pltpu.async_copy(den_hbm.at[dbuf], dng, sem).wait()
        pltpu.async_copy(xw_hbm.at[sbuf], xwg, sem).wait()

        def edge(e, carry2):
            ex = _edge_ex(asg[e, :], adg[e, :], shift, maskv, gv)
            w16 = ex / (dng[e, :] + 1e-16) * INVH
            wv = [w16.at[hidx[h]].get(mode="promise_in_bounds")
                  for h in range(H)]
            for cb in range(C // 16):
                a = wv[0] * xwg[e, pl.ds(cb * 16, 16)]
                for h in range(1, H):
                    a = a + wv[h] * xwg[e, pl.ds(h * C + cb * 16, 16)]
                msg[e, pl.ds(cb * 16, 16)] = a
            return carry2

        lax.fori_loop(0, CHB, edge, 0)
        pltpu.sync_copy(msg, acc.at[dbuf], add=True)
        return carry

    lax.fori_loop(0, NCHB, chunk, 0)
    plsc.subcore_barrier()
    for k in range(NZB):
        pltpu.sync_copy(acc.at[pl.ds(sid * RPT + k * RCHB, RCHB)], zb)
        pltpu.sync_copy(zb, pout_hbm.at[cid, pl.ds(sid * RPT + k * RCHB, RCHB)])

    @pl.when(sid == 15)
    def _():
        pltpu.sync_copy(acc.at[pl.ds(16 * RPT, 16)], zb.at[pl.ds(0, 16)])
        pltpu.sync_copy(zb.at[pl.ds(0, 16)],
                        pout_hbm.at[cid, pl.ds(16 * RPT, 16)])


def _pass_b(alphas, g16, den, xw, src, dst):
    kfn = pl.kernel(
        _pb_body,
        out_type=jax.ShapeDtypeStruct((2, N, C), jnp.float32),
        mesh=plsc.VectorSubcoreMesh(core_axis_name="c", subcore_axis_name="s"),
        compiler_params=pltpu.CompilerParams(use_tc_tiling_on_sc=False),
        scratch_types=[
            pltpu.VMEM((HP,), jnp.float32),
            pltpu.VMEM((CHB,), jnp.int32),
            pltpu.VMEM((CHB,), jnp.int32),
            pltpu.VMEM((CHB, HP), jnp.float32),
            pltpu.VMEM((CHB, HP), jnp.float32),
            pltpu.VMEM((CHB, HP), jnp.float32),
            pltpu.VMEM((CHB, H * C), jnp.float32),
            pltpu.VMEM((CHB, C), jnp.float32),
            pltpu.VMEM((RCHB, C), jnp.float32),
            pltpu.VMEM_SHARED((N, C), jnp.float32),
            pltpu.SemaphoreType.DMA,
        ],
    )
    return kfn(alphas, g16, den, xw, src, dst)


def kernel(features, edge_index, W, att_src, att_dst, bias):
    f32 = jnp.float32
    eye5 = jnp.eye(H, dtype=f32)
    src_cols = (att_src[:, :, None] * eye5[:, None, :]).reshape(H * C, H)
    dst_cols = (att_dst[:, :, None] * eye5[:, None, :]).reshape(H * C, H)
    zpad = jnp.zeros((H * C, 3), f32)
    attm = jnp.concatenate([src_cols, zpad, dst_cols, zpad], axis=1)
    psum = jnp.zeros((HP, HP), f32)
    idx = jnp.arange(H)
    psum = psum.at[idx, idx].set(1.0).at[idx + 8, idx].set(1.0)
    mask = jnp.concatenate(
        [jnp.ones((1, H), f32), jnp.zeros((1, HP - H), f32)], axis=1)
    r = jnp.concatenate(
        [jnp.repeat(jnp.eye(H, dtype=f32), C, axis=1),
         jnp.zeros((HP - H, H * C), f32)], axis=0)

    src = edge_index[0]
    dst = edge_index[1]

    alphas, exs, g16 = _s1(features, W, attm, psum, mask)
    xw = _s2(features, W)
    pa = _pass_a(alphas, g16.reshape(HP), src, dst)
    den, selfw = _s3(pa, exs)
    pb = _pass_b(alphas, g16.reshape(HP), den, xw, src, dst)
    return _s4(features, pb[0], pb[1], xw, selfw, r, bias.reshape(1, C))
